# baseline (speedup 1.0000x reference)
import functools
import jax
import jax.numpy as jnp
from jax import lax
from jax.experimental import pallas as pl
from jax.experimental.pallas import tpu as pltpu
from jax.experimental.pallas import tpu_sc as plsc

_D = 64
_NC = 2
_NS = 16
_NW = _NC * _NS
_LANES = 16


@functools.partial(jax.jit, static_argnums=(2, 3))
def _emb_lookup(idx_flat, table, num_idx, chunk):
  per_w = num_idx // _NW
  n_chunks = per_w // chunk
  assert per_w % chunk == 0 and chunk % 8 == 0

  mesh = plsc.VectorSubcoreMesh(
      core_axis_name="c", subcore_axis_name="s",
      num_cores=_NC, num_subcores=_NS)

  @functools.partial(
      pl.kernel,
      out_type=jax.ShapeDtypeStruct((num_idx, _D), jnp.float32),
      mesh=mesh,
      compiler_params=pltpu.CompilerParams(use_tc_tiling_on_sc=False),
      scratch_types=[
          pltpu.VMEM((per_w,), jnp.int32),
          pltpu.VMEM((2, chunk, _D), jnp.float32),
          pltpu.SemaphoreType.DMA,
          pltpu.SemaphoreType.DMA,
      ],
  )
  def k(idx_hbm, table_hbm, out_hbm, idx_v, rows_v, sem0, sem1):
    wid = lax.axis_index("s") * _NC + lax.axis_index("c")
    base = pl.multiple_of(wid * per_w, chunk)
    pltpu.sync_copy(idx_hbm.at[pl.ds(base, per_w)], idx_v)

    sems = (sem0, sem1)

    def start_gather(c, b):
      idx_slice = idx_v.at[pl.ds(pl.multiple_of(c * chunk, chunk), chunk)]
      pltpu.async_copy(table_hbm.at[idx_slice], rows_v.at[b], sems[b])

    start_gather(0, 0)
    start_gather(1, 1)

    def pair_body(p, _):
      for b in range(2):
        c = p * 2 + b
        buf = rows_v.at[b]
        pltpu.make_async_copy(table_hbm.at[pl.ds(0, chunk)],
                              buf, sems[b]).wait()

        @pl.loop(0, chunk)
        def _scale(r):
          for kk in range(_D // _LANES):
            sl = pl.ds(kk * _LANES, _LANES)
            buf[r, sl] = buf[r, sl] * 8.0

        dst = out_hbm.at[pl.ds(pl.multiple_of(base + c * chunk, chunk), chunk)]
        pltpu.sync_copy(buf, dst)

        @pl.when(c + 2 < n_chunks)
        def _():
          start_gather(c + 2, b)
      return ()

    lax.fori_loop(0, n_chunks // 2, pair_body, ())

  return k(idx_flat, table)


def kernel(x, emb_weight):
  b, t = x.shape
  idx_flat = x.reshape(b * t).astype(jnp.int32)
  out = _emb_lookup(idx_flat, emb_weight, b * t, 512)
  return out.reshape(b, t, _D)
